# baseline (speedup 1.0000x reference)
import functools
import jax
import jax.numpy as jnp
from jax import lax
from jax.experimental import pallas as pl
from jax.experimental.pallas import tpu as pltpu
from jax.experimental.pallas import tpu_sc as plsc

_NP = 10240
_NPH = 10368
_CS = 1280
_B = 128
_BSH = 7
_ETILE = 1024
_RING = 16
_EP = 163840

_MESH = plsc.VectorSubcoreMesh(core_axis_name="c", subcore_axis_name="s")
_SC_PARAMS = pltpu.CompilerParams(needs_layout_passes=False)


def _deg_body(dst2_hbm, zeros_hbm, ones_hbm, out_hbm, dacc, dtile, onesv):
    core = lax.axis_index("c")
    sub = lax.axis_index("s")
    rows_total = dst2_hbm.shape[0]
    r_sub = rows_total // 32
    stripe = _NPH // 16
    pltpu.sync_copy(zeros_hbm.at[pl.ds(sub * stripe, stripe), :],
                    dacc.at[pl.ds(sub * stripe, stripe), :])
    pltpu.sync_copy(ones_hbm, onesv)
    plsc.subcore_barrier()
    rbase = (core * 16 + sub) * r_sub
    trows = dtile.shape[0]
    n_tiles = r_sub // trows

    @pl.loop(0, n_tiles)
    def _(t):
        pltpu.sync_copy(dst2_hbm.at[pl.ds(rbase + t * trows, trows), :], dtile)

        @pl.loop(0, trows)
        def _(bk):
            pltpu.sync_copy(onesv, dacc.at[dtile.at[bk]], add=True)

    plsc.subcore_barrier()
    pltpu.sync_copy(dacc.at[pl.ds(sub * stripe, stripe), :],
                    out_hbm.at[core, pl.ds(sub * stripe, stripe), :])


def _deg(dst2):
    zeros = jnp.zeros((_NPH, 128), jnp.float32)
    ones = jnp.zeros((_B, 128), jnp.float32).at[:, 0].set(1.0)
    f = pl.kernel(
        _deg_body,
        out_type=jax.ShapeDtypeStruct((2, _NPH, 128), jnp.float32),
        mesh=_MESH,
        scratch_types=[
            pltpu.VMEM_SHARED((_NPH, 128), jnp.float32),
            pltpu.VMEM((8, _B), jnp.int32),
            pltpu.VMEM((_B, 128), jnp.float32),
        ],
        compiler_params=_SC_PARAMS,
    )
    return f(dst2, zeros, ones)


def _agg_body(g_hbm, src_hbm, dst_hbm, out_hbm,
              acc, stile, dtile, srcb, dstb, rows):
    core = lax.axis_index("c")
    sub = lax.axis_index("s")
    e_sub = src_hbm.shape[0] // 16
    n_tiles = e_sub // _ETILE
    n_vecs = _ETILE // 16
    rstripe = _CS // 16
    iota16 = lax.iota(jnp.int32, 16)

    for p in range(_NP // _CS // 2):
        chunk = core + 2 * p
        base = chunk * _CS
        pltpu.sync_copy(g_hbm.at[pl.ds(base + sub * rstripe, rstripe)],
                        acc.at[pl.ds(sub * rstripe, rstripe)])
        plsc.subcore_barrier()

        ebase = sub * e_sub

        def batch_body(j, carry):
            jr = j & (_RING - 1)
            pltpu.sync_copy(g_hbm.at[srcb.at[jr]], rows)
            pltpu.sync_copy(rows, acc.at[dstb.at[jr]], add=True)
            return carry

        def tile_body(t, carry):
            cnt, jdone = carry
            pltpu.sync_copy(src_hbm.at[pl.ds(ebase + t * _ETILE, _ETILE)], stile)
            pltpu.sync_copy(dst_hbm.at[pl.ds(ebase + t * _ETILE, _ETILE)], dtile)

            def vec_body(v, cnt):
                s16 = stile[pl.ds(v * 16, 16)]
                d16 = dtile[pl.ds(v * 16, 16)]
                dloc = d16 - base
                m = (dloc >= 0) & (dloc < _CS)
                mi = m.astype(jnp.int32)
                pos = cnt + plsc.cumsum(mi) - 1
                plsc.store_scatter(
                    srcb, [(pos >> _BSH) & (_RING - 1), pos & (_B - 1)],
                    s16, mask=m)
                plsc.store_scatter(
                    dstb, [(pos >> _BSH) & (_RING - 1), pos & (_B - 1)],
                    dloc, mask=m)
                return cnt + jnp.sum(mi)

            cnt = lax.fori_loop(0, n_vecs, vec_body, cnt)
            jdone = lax.fori_loop(jdone, cnt >> _BSH, batch_body, jdone)
            return cnt, cnt >> _BSH

        cnt, jdone = lax.fori_loop(0, n_tiles, tile_body,
                                   (jnp.int32(0), jnp.int32(0)))

        for k in range(_B // 16):
            posk = cnt + k * 16 + iota16
            plsc.store_scatter(
                srcb, [(posk >> _BSH) & (_RING - 1), posk & (_B - 1)],
                jnp.zeros((16,), jnp.int32))
            plsc.store_scatter(
                dstb, [(posk >> _BSH) & (_RING - 1), posk & (_B - 1)],
                jnp.full((16,), _CS, jnp.int32))

        nb = (cnt + (_B - 1)) >> _BSH
        lax.fori_loop(jdone, nb, batch_body, jdone)
        plsc.subcore_barrier()
        pltpu.sync_copy(acc.at[pl.ds(sub * rstripe, rstripe)],
                        out_hbm.at[pl.ds(base + sub * rstripe, rstripe)])


def _agg(g, src, dst):
    f = pl.kernel(
        _agg_body,
        out_type=jax.ShapeDtypeStruct((_NP, 4, 128), jnp.float32),
        mesh=_MESH,
        scratch_types=[
            pltpu.VMEM_SHARED((_CS + 8, 4, 128), jnp.float32),
            pltpu.VMEM((_ETILE,), jnp.int32),
            pltpu.VMEM((_ETILE,), jnp.int32),
            pltpu.VMEM((_RING, _B), jnp.int32),
            pltpu.VMEM((_RING, _B), jnp.int32),
            pltpu.VMEM((_B, 4, 128), jnp.float32),
        ],
        compiler_params=_SC_PARAMS,
    )
    return f(g.reshape(_NP, 4, 128), src, dst).reshape(_NP, 512)


def _matmul_body(x_ref, w_ref, out_ref):
    out_ref[...] = jax.lax.dot_general(
        x_ref[...], w_ref[...], (((1,), (0,)), ((), ())),
        preferred_element_type=jnp.float32,
        precision=jax.lax.Precision.HIGHEST,
    )


def _matmul(x, W):
    n, d_in = x.shape
    d_out = W.shape[1]
    blk = 2048
    return pl.pallas_call(
        _matmul_body,
        grid=(n // blk,),
        in_specs=[
            pl.BlockSpec((blk, d_in), lambda i: (i, 0)),
            pl.BlockSpec((d_in, d_out), lambda i: (0, 0)),
        ],
        out_specs=pl.BlockSpec((blk, d_out), lambda i: (i, 0)),
        out_shape=jax.ShapeDtypeStruct((n, d_out), jnp.float32),
    )(x, W)


def _scale_body(h_ref, parts_ref, g_ref, dinv_ref):
    p = parts_ref[...]
    deg = p[0, :, 0] + p[1, :, 0] + 1.0
    dinv = jax.lax.rsqrt(deg).reshape(-1, 1)
    dinv_ref[...] = dinv
    g_ref[...] = h_ref[...] * dinv


def _scale(h, parts):
    n, d = h.shape
    blk = 2048
    return pl.pallas_call(
        _scale_body,
        grid=(n // blk,),
        in_specs=[
            pl.BlockSpec((blk, d), lambda i: (i, 0)),
            pl.BlockSpec((2, blk, 128), lambda i: (0, i, 0)),
        ],
        out_specs=[
            pl.BlockSpec((blk, d), lambda i: (i, 0)),
            pl.BlockSpec((blk, 1), lambda i: (i, 0)),
        ],
        out_shape=[
            jax.ShapeDtypeStruct((n, d), jnp.float32),
            jax.ShapeDtypeStruct((n, 1), jnp.float32),
        ],
    )(h, parts)


def _bn_stats_body(acc_ref, dinv_ref, b_ref, stats_ref):
    i = pl.program_id(0)
    y = acc_ref[...] * dinv_ref[...] + b_ref[...].reshape(1, -1)
    s = jnp.sum(y, axis=0, keepdims=True)
    ss = jnp.sum(y * y, axis=0, keepdims=True)
    blk_stats = jnp.concatenate([s, ss], axis=0)

    @pl.when(i == 0)
    def _():
        stats_ref[...] = blk_stats

    @pl.when(i > 0)
    def _():
        stats_ref[...] += blk_stats


def _bn_apply_body(acc_ref, dinv_ref, b_ref, gamma_ref, beta_ref, stats_ref,
                   out_ref, *, n):
    y = acc_ref[...] * dinv_ref[...] + b_ref[...].reshape(1, -1)
    mean = stats_ref[0, :].reshape(1, -1) / n
    var = stats_ref[1, :].reshape(1, -1) / n - mean * mean
    z = (y - mean) * jax.lax.rsqrt(var + 1e-5)
    out_ref[...] = jnp.maximum(
        gamma_ref[...].reshape(1, -1) * z + beta_ref[...].reshape(1, -1), 0.0
    )


def _bn_relu(agg, dinv, b, gamma, beta, n):
    d = agg.shape[1]
    blk = 2000
    grid = (n // blk,)
    stats = pl.pallas_call(
        _bn_stats_body,
        grid=grid,
        in_specs=[
            pl.BlockSpec((blk, d), lambda i: (i, 0)),
            pl.BlockSpec((blk, 1), lambda i: (i, 0)),
            pl.BlockSpec((d,), lambda i: (0,)),
        ],
        out_specs=pl.BlockSpec((2, d), lambda i: (0, 0)),
        out_shape=jax.ShapeDtypeStruct((2, d), jnp.float32),
    )(agg, dinv, b)
    return pl.pallas_call(
        functools.partial(_bn_apply_body, n=float(n)),
        grid=grid,
        in_specs=[
            pl.BlockSpec((blk, d), lambda i: (i, 0)),
            pl.BlockSpec((blk, 1), lambda i: (i, 0)),
            pl.BlockSpec((d,), lambda i: (0,)),
            pl.BlockSpec((d,), lambda i: (0,)),
            pl.BlockSpec((d,), lambda i: (0,)),
            pl.BlockSpec((2, d), lambda i: (0, 0)),
        ],
        out_specs=pl.BlockSpec((blk, d), lambda i: (i, 0)),
        out_shape=jax.ShapeDtypeStruct((n, d), jnp.float32),
    )(agg, dinv, b, gamma, beta, stats)


@jax.jit
def kernel(x, edge_index, W, b, gamma, beta):
    n = x.shape[0]
    e = edge_index.shape[1]
    src = jnp.pad(edge_index[0], (0, _EP - e))
    dst = jnp.pad(edge_index[1], (0, _EP - e), constant_values=_NP)
    xp = jnp.pad(x, ((0, _NP - n), (0, 0)))
    parts = _deg(dst.reshape(_EP // _B, _B))
    h = _matmul(xp, W)
    g, dinv = _scale(h, parts)
    aggp = _agg(g, src, dst)
    return _bn_relu(aggp, dinv, b, gamma, beta, n)


# speedup vs baseline: 1.5677x; 1.5677x over previous
import functools
import jax
import jax.numpy as jnp
from jax import lax
from jax.experimental import pallas as pl
from jax.experimental.pallas import tpu as pltpu
from jax.experimental.pallas import tpu_sc as plsc

_NP = 10240
_NPH = 10368
_CS = 2560
_B = 64
_BSH = 6
_ETILE = 1024
_RING = 32
_EP = 163840

_MESH = plsc.VectorSubcoreMesh(core_axis_name="c", subcore_axis_name="s")
_SC_PARAMS = pltpu.CompilerParams(needs_layout_passes=False)


def _deg_body(dst2_hbm, zeros_hbm, ones_hbm, out_hbm, dacc, dtile, onesv):
    core = lax.axis_index("c")
    sub = lax.axis_index("s")
    rows_total = dst2_hbm.shape[0]
    r_sub = rows_total // 32
    stripe = _NPH // 16
    pltpu.sync_copy(zeros_hbm.at[pl.ds(sub * stripe, stripe), :],
                    dacc.at[pl.ds(sub * stripe, stripe), :])
    pltpu.sync_copy(ones_hbm, onesv)
    plsc.subcore_barrier()
    rbase = (core * 16 + sub) * r_sub
    trows = dtile.shape[0]
    n_tiles = r_sub // trows

    @pl.loop(0, n_tiles)
    def _(t):
        pltpu.sync_copy(dst2_hbm.at[pl.ds(rbase + t * trows, trows), :], dtile)

        @pl.loop(0, trows)
        def _(bk):
            pltpu.sync_copy(onesv, dacc.at[dtile.at[bk]], add=True)

    plsc.subcore_barrier()
    pltpu.sync_copy(dacc.at[pl.ds(sub * stripe, stripe), :],
                    out_hbm.at[core, pl.ds(sub * stripe, stripe), :])


def _deg(dst2):
    zeros = jnp.zeros((_NPH, 128), jnp.float32)
    ones = jnp.zeros((_B, 128), jnp.float32).at[:, 0].set(1.0)
    f = pl.kernel(
        _deg_body,
        out_type=jax.ShapeDtypeStruct((2, _NPH, 128), jnp.float32),
        mesh=_MESH,
        scratch_types=[
            pltpu.VMEM_SHARED((_NPH, 128), jnp.float32),
            pltpu.VMEM((8, _B), jnp.int32),
            pltpu.VMEM((_B, 128), jnp.float32),
        ],
        compiler_params=_SC_PARAMS,
    )
    return f(dst2, zeros, ones)


def _agg_body(g_hbm, src_hbm, dst_hbm, out_hbm,
              acc, stile, dtile, srcb, dstb, rows):
    core = lax.axis_index("c")
    sub = lax.axis_index("s")
    e_sub = src_hbm.shape[0] // 16
    n_tiles = e_sub // _ETILE
    n_vecs = _ETILE // 16
    rstripe = _CS // 16
    iota16 = lax.iota(jnp.int32, 16)

    for p in range(_NP // _CS // 2):
        chunk = core + 2 * p
        base = chunk * _CS
        pltpu.sync_copy(g_hbm.at[pl.ds(base + sub * rstripe, rstripe)],
                        acc.at[pl.ds(sub * rstripe, rstripe)])
        plsc.subcore_barrier()

        ebase = sub * e_sub

        def batch_body(j, carry):
            jr = j & (_RING - 1)
            pltpu.sync_copy(g_hbm.at[srcb.at[jr]], rows)
            pltpu.sync_copy(rows, acc.at[dstb.at[jr]], add=True)
            return carry

        def tile_body(t, carry):
            cnt, jdone = carry
            pltpu.sync_copy(src_hbm.at[pl.ds(ebase + t * _ETILE, _ETILE)], stile)
            pltpu.sync_copy(dst_hbm.at[pl.ds(ebase + t * _ETILE, _ETILE)], dtile)

            def vec_body(v, cnt):
                s16 = stile[pl.ds(v * 16, 16)]
                d16 = dtile[pl.ds(v * 16, 16)]
                dloc = d16 - base
                m = (dloc >= 0) & (dloc < _CS)
                mi = m.astype(jnp.int32)
                pos = cnt + plsc.cumsum(mi) - 1
                plsc.store_scatter(
                    srcb, [(pos >> _BSH) & (_RING - 1), pos & (_B - 1)],
                    s16, mask=m)
                plsc.store_scatter(
                    dstb, [(pos >> _BSH) & (_RING - 1), pos & (_B - 1)],
                    dloc, mask=m)
                return cnt + jnp.sum(mi)

            cnt = lax.fori_loop(0, n_vecs, vec_body, cnt)
            jdone = lax.fori_loop(jdone, cnt >> _BSH, batch_body, jdone)
            return cnt, cnt >> _BSH

        cnt, jdone = lax.fori_loop(0, n_tiles, tile_body,
                                   (jnp.int32(0), jnp.int32(0)))

        for k in range(_B // 16):
            posk = cnt + k * 16 + iota16
            plsc.store_scatter(
                srcb, [(posk >> _BSH) & (_RING - 1), posk & (_B - 1)],
                jnp.zeros((16,), jnp.int32))
            plsc.store_scatter(
                dstb, [(posk >> _BSH) & (_RING - 1), posk & (_B - 1)],
                jnp.full((16,), _CS, jnp.int32))

        nb = (cnt + (_B - 1)) >> _BSH
        lax.fori_loop(jdone, nb, batch_body, jdone)
        plsc.subcore_barrier()
        pltpu.sync_copy(acc.at[pl.ds(sub * rstripe, rstripe)],
                        out_hbm.at[pl.ds(base + sub * rstripe, rstripe)])


def _agg(g, src, dst):
    f = pl.kernel(
        _agg_body,
        out_type=jax.ShapeDtypeStruct((_NP, 4, 128), jnp.float32),
        mesh=_MESH,
        scratch_types=[
            pltpu.VMEM_SHARED((_CS + 8, 4, 128), jnp.float32),
            pltpu.VMEM((_ETILE,), jnp.int32),
            pltpu.VMEM((_ETILE,), jnp.int32),
            pltpu.VMEM((_RING, _B), jnp.int32),
            pltpu.VMEM((_RING, _B), jnp.int32),
            pltpu.VMEM((_B, 4, 128), jnp.float32),
        ],
        compiler_params=_SC_PARAMS,
    )
    return f(g.reshape(_NP, 4, 128), src, dst).reshape(_NP, 512)


def _matmul_body(x_ref, w_ref, out_ref):
    out_ref[...] = jax.lax.dot_general(
        x_ref[...], w_ref[...], (((1,), (0,)), ((), ())),
        preferred_element_type=jnp.float32,
        precision=jax.lax.Precision.HIGHEST,
    )


def _matmul(x, W):
    n, d_in = x.shape
    d_out = W.shape[1]
    blk = 2048
    return pl.pallas_call(
        _matmul_body,
        grid=(n // blk,),
        in_specs=[
            pl.BlockSpec((blk, d_in), lambda i: (i, 0)),
            pl.BlockSpec((d_in, d_out), lambda i: (0, 0)),
        ],
        out_specs=pl.BlockSpec((blk, d_out), lambda i: (i, 0)),
        out_shape=jax.ShapeDtypeStruct((n, d_out), jnp.float32),
    )(x, W)


def _scale_body(h_ref, parts_ref, g_ref, dinv_ref):
    p = parts_ref[...]
    deg = p[0, :, 0] + p[1, :, 0] + 1.0
    dinv = jax.lax.rsqrt(deg).reshape(-1, 1)
    dinv_ref[...] = dinv
    g_ref[...] = h_ref[...] * dinv


def _scale(h, parts):
    n, d = h.shape
    blk = 2048
    return pl.pallas_call(
        _scale_body,
        grid=(n // blk,),
        in_specs=[
            pl.BlockSpec((blk, d), lambda i: (i, 0)),
            pl.BlockSpec((2, blk, 128), lambda i: (0, i, 0)),
        ],
        out_specs=[
            pl.BlockSpec((blk, d), lambda i: (i, 0)),
            pl.BlockSpec((blk, 1), lambda i: (i, 0)),
        ],
        out_shape=[
            jax.ShapeDtypeStruct((n, d), jnp.float32),
            jax.ShapeDtypeStruct((n, 1), jnp.float32),
        ],
    )(h, parts)


def _bn_stats_body(acc_ref, dinv_ref, b_ref, stats_ref):
    i = pl.program_id(0)
    y = acc_ref[...] * dinv_ref[...] + b_ref[...].reshape(1, -1)
    s = jnp.sum(y, axis=0, keepdims=True)
    ss = jnp.sum(y * y, axis=0, keepdims=True)
    blk_stats = jnp.concatenate([s, ss], axis=0)

    @pl.when(i == 0)
    def _():
        stats_ref[...] = blk_stats

    @pl.when(i > 0)
    def _():
        stats_ref[...] += blk_stats


def _bn_apply_body(acc_ref, dinv_ref, b_ref, gamma_ref, beta_ref, stats_ref,
                   out_ref, *, n):
    y = acc_ref[...] * dinv_ref[...] + b_ref[...].reshape(1, -1)
    mean = stats_ref[0, :].reshape(1, -1) / n
    var = stats_ref[1, :].reshape(1, -1) / n - mean * mean
    z = (y - mean) * jax.lax.rsqrt(var + 1e-5)
    out_ref[...] = jnp.maximum(
        gamma_ref[...].reshape(1, -1) * z + beta_ref[...].reshape(1, -1), 0.0
    )


def _bn_relu(agg, dinv, b, gamma, beta, n):
    d = agg.shape[1]
    blk = 2000
    grid = (n // blk,)
    stats = pl.pallas_call(
        _bn_stats_body,
        grid=grid,
        in_specs=[
            pl.BlockSpec((blk, d), lambda i: (i, 0)),
            pl.BlockSpec((blk, 1), lambda i: (i, 0)),
            pl.BlockSpec((d,), lambda i: (0,)),
        ],
        out_specs=pl.BlockSpec((2, d), lambda i: (0, 0)),
        out_shape=jax.ShapeDtypeStruct((2, d), jnp.float32),
    )(agg, dinv, b)
    return pl.pallas_call(
        functools.partial(_bn_apply_body, n=float(n)),
        grid=grid,
        in_specs=[
            pl.BlockSpec((blk, d), lambda i: (i, 0)),
            pl.BlockSpec((blk, 1), lambda i: (i, 0)),
            pl.BlockSpec((d,), lambda i: (0,)),
            pl.BlockSpec((d,), lambda i: (0,)),
            pl.BlockSpec((d,), lambda i: (0,)),
            pl.BlockSpec((2, d), lambda i: (0, 0)),
        ],
        out_specs=pl.BlockSpec((blk, d), lambda i: (i, 0)),
        out_shape=jax.ShapeDtypeStruct((n, d), jnp.float32),
    )(agg, dinv, b, gamma, beta, stats)


@jax.jit
def kernel(x, edge_index, W, b, gamma, beta):
    n = x.shape[0]
    e = edge_index.shape[1]
    src = jnp.pad(edge_index[0], (0, _EP - e))
    dst = jnp.pad(edge_index[1], (0, _EP - e), constant_values=_NP)
    xp = jnp.pad(x, ((0, _NP - n), (0, 0)))
    parts = _deg(dst.reshape(_EP // _B, _B))
    h = _matmul(xp, W)
    g, dinv = _scale(h, parts)
    aggp = _agg(g, src, dst)
    return _bn_relu(aggp, dinv, b, gamma, beta, n)


# speedup vs baseline: 1.5964x; 1.0183x over previous
import functools
import jax
import jax.numpy as jnp
from jax import lax
from jax.experimental import pallas as pl
from jax.experimental.pallas import tpu as pltpu
from jax.experimental.pallas import tpu_sc as plsc

_NP = 10240
_NPH = 10368
_CS = 2560
_B = 64
_BSH = 6
_ETILE = 1024
_RING = 32
_EP = 163840

_MESH = plsc.VectorSubcoreMesh(core_axis_name="c", subcore_axis_name="s")
_SC_PARAMS = pltpu.CompilerParams(needs_layout_passes=False)


def _deg_body(dst_hbm, zeros_hbm, out_hbm, hist, dtile):
    core = lax.axis_index("c")
    sub = lax.axis_index("s")
    wid = core * 16 + sub
    e_sub = dst_hbm.shape[0] // 32
    n_tiles = e_sub // _ETILE
    ones16 = jnp.full((16,), 1.0, jnp.float32)
    ebase = wid * e_sub
    pltpu.sync_copy(zeros_hbm, hist)

    @pl.loop(0, n_tiles)
    def _(t):
        pltpu.sync_copy(dst_hbm.at[pl.ds(ebase + t * _ETILE, _ETILE)], dtile)

        @pl.loop(0, _ETILE // 16)
        def _(v):
            plsc.addupdate_scatter(hist, [dtile[pl.ds(v * 16, 16)]], ones16)

    pltpu.sync_copy(hist, out_hbm.at[wid])


def _deg(dst):
    zeros = jnp.zeros((_NPH,), jnp.float32)
    f = pl.kernel(
        _deg_body,
        out_type=jax.ShapeDtypeStruct((32, _NPH), jnp.float32),
        mesh=_MESH,
        scratch_types=[
            pltpu.VMEM((_NPH,), jnp.float32),
            pltpu.VMEM((_ETILE,), jnp.int32),
        ],
        compiler_params=_SC_PARAMS,
    )
    return f(dst, zeros)


def _agg_body(g_hbm, src_hbm, dst_hbm, out_hbm,
              acc, stile, dtile, srcb, dstb, rows):
    core = lax.axis_index("c")
    sub = lax.axis_index("s")
    e_sub = src_hbm.shape[0] // 16
    n_tiles = e_sub // _ETILE
    n_vecs = _ETILE // 16
    rstripe = _CS // 16
    iota16 = lax.iota(jnp.int32, 16)

    for p in range(_NP // _CS // 2):
        chunk = core + 2 * p
        base = chunk * _CS
        pltpu.sync_copy(g_hbm.at[pl.ds(base + sub * rstripe, rstripe)],
                        acc.at[pl.ds(sub * rstripe, rstripe)])
        plsc.subcore_barrier()

        ebase = sub * e_sub

        def batch_body(j, carry):
            jr = j & (_RING - 1)
            pltpu.sync_copy(g_hbm.at[srcb.at[jr]], rows)
            pltpu.sync_copy(rows, acc.at[dstb.at[jr]], add=True)
            return carry

        def tile_body(t, carry):
            cnt, jdone = carry
            pltpu.sync_copy(src_hbm.at[pl.ds(ebase + t * _ETILE, _ETILE)], stile)
            pltpu.sync_copy(dst_hbm.at[pl.ds(ebase + t * _ETILE, _ETILE)], dtile)

            def vec_body(v, cnt):
                s16 = stile[pl.ds(v * 16, 16)]
                d16 = dtile[pl.ds(v * 16, 16)]
                dloc = d16 - base
                m = (dloc >= 0) & (dloc < _CS)
                mi = m.astype(jnp.int32)
                pos = cnt + plsc.cumsum(mi) - 1
                plsc.store_scatter(
                    srcb, [(pos >> _BSH) & (_RING - 1), pos & (_B - 1)],
                    s16, mask=m)
                plsc.store_scatter(
                    dstb, [(pos >> _BSH) & (_RING - 1), pos & (_B - 1)],
                    dloc, mask=m)
                return cnt + jnp.sum(mi)

            cnt = lax.fori_loop(0, n_vecs, vec_body, cnt)
            jdone = lax.fori_loop(jdone, cnt >> _BSH, batch_body, jdone)
            return cnt, cnt >> _BSH

        cnt, jdone = lax.fori_loop(0, n_tiles, tile_body,
                                   (jnp.int32(0), jnp.int32(0)))

        for k in range(_B // 16):
            posk = cnt + k * 16 + iota16
            plsc.store_scatter(
                srcb, [(posk >> _BSH) & (_RING - 1), posk & (_B - 1)],
                jnp.zeros((16,), jnp.int32))
            plsc.store_scatter(
                dstb, [(posk >> _BSH) & (_RING - 1), posk & (_B - 1)],
                jnp.full((16,), _CS, jnp.int32))

        nb = (cnt + (_B - 1)) >> _BSH
        lax.fori_loop(jdone, nb, batch_body, jdone)
        plsc.subcore_barrier()
        pltpu.sync_copy(acc.at[pl.ds(sub * rstripe, rstripe)],
                        out_hbm.at[pl.ds(base + sub * rstripe, rstripe)])


def _agg(g, src, dst):
    f = pl.kernel(
        _agg_body,
        out_type=jax.ShapeDtypeStruct((_NP, 4, 128), jnp.float32),
        mesh=_MESH,
        scratch_types=[
            pltpu.VMEM_SHARED((_CS + 8, 4, 128), jnp.float32),
            pltpu.VMEM((_ETILE,), jnp.int32),
            pltpu.VMEM((_ETILE,), jnp.int32),
            pltpu.VMEM((_RING, _B), jnp.int32),
            pltpu.VMEM((_RING, _B), jnp.int32),
            pltpu.VMEM((_B, 4, 128), jnp.float32),
        ],
        compiler_params=_SC_PARAMS,
    )
    return f(g.reshape(_NP, 4, 128), src, dst).reshape(_NP, 512)


def _matmul_body(x_ref, w_ref, out_ref):
    out_ref[...] = jax.lax.dot_general(
        x_ref[...], w_ref[...], (((1,), (0,)), ((), ())),
        preferred_element_type=jnp.float32,
        precision=jax.lax.Precision.HIGHEST,
    )


def _matmul(x, W):
    n, d_in = x.shape
    d_out = W.shape[1]
    blk = 2048
    return pl.pallas_call(
        _matmul_body,
        grid=(n // blk,),
        in_specs=[
            pl.BlockSpec((blk, d_in), lambda i: (i, 0)),
            pl.BlockSpec((d_in, d_out), lambda i: (0, 0)),
        ],
        out_specs=pl.BlockSpec((blk, d_out), lambda i: (i, 0)),
        out_shape=jax.ShapeDtypeStruct((n, d_out), jnp.float32),
    )(x, W)


def _scale_body(h_ref, parts_ref, g_ref, dinv_ref):
    deg = jnp.sum(parts_ref[...], axis=0) + 1.0
    dinv = jax.lax.rsqrt(deg).reshape(-1, 1)
    dinv_ref[...] = dinv
    g_ref[...] = h_ref[...] * dinv


def _scale(h, parts):
    n, d = h.shape
    blk = 2048
    return pl.pallas_call(
        _scale_body,
        grid=(n // blk,),
        in_specs=[
            pl.BlockSpec((blk, d), lambda i: (i, 0)),
            pl.BlockSpec((32, blk), lambda i: (0, i)),
        ],
        out_specs=[
            pl.BlockSpec((blk, d), lambda i: (i, 0)),
            pl.BlockSpec((blk, 1), lambda i: (i, 0)),
        ],
        out_shape=[
            jax.ShapeDtypeStruct((n, d), jnp.float32),
            jax.ShapeDtypeStruct((n, 1), jnp.float32),
        ],
    )(h, parts)


def _bn_stats_body(acc_ref, dinv_ref, b_ref, stats_ref):
    i = pl.program_id(0)
    y = acc_ref[...] * dinv_ref[...] + b_ref[...].reshape(1, -1)
    s = jnp.sum(y, axis=0, keepdims=True)
    ss = jnp.sum(y * y, axis=0, keepdims=True)
    blk_stats = jnp.concatenate([s, ss], axis=0)

    @pl.when(i == 0)
    def _():
        stats_ref[...] = blk_stats

    @pl.when(i > 0)
    def _():
        stats_ref[...] += blk_stats


def _bn_apply_body(acc_ref, dinv_ref, b_ref, gamma_ref, beta_ref, stats_ref,
                   out_ref, *, n):
    y = acc_ref[...] * dinv_ref[...] + b_ref[...].reshape(1, -1)
    mean = stats_ref[0, :].reshape(1, -1) / n
    var = stats_ref[1, :].reshape(1, -1) / n - mean * mean
    z = (y - mean) * jax.lax.rsqrt(var + 1e-5)
    out_ref[...] = jnp.maximum(
        gamma_ref[...].reshape(1, -1) * z + beta_ref[...].reshape(1, -1), 0.0
    )


def _bn_relu(agg, dinv, b, gamma, beta, n):
    d = agg.shape[1]
    blk = 2000
    grid = (n // blk,)
    stats = pl.pallas_call(
        _bn_stats_body,
        grid=grid,
        in_specs=[
            pl.BlockSpec((blk, d), lambda i: (i, 0)),
            pl.BlockSpec((blk, 1), lambda i: (i, 0)),
            pl.BlockSpec((d,), lambda i: (0,)),
        ],
        out_specs=pl.BlockSpec((2, d), lambda i: (0, 0)),
        out_shape=jax.ShapeDtypeStruct((2, d), jnp.float32),
    )(agg, dinv, b)
    return pl.pallas_call(
        functools.partial(_bn_apply_body, n=float(n)),
        grid=grid,
        in_specs=[
            pl.BlockSpec((blk, d), lambda i: (i, 0)),
            pl.BlockSpec((blk, 1), lambda i: (i, 0)),
            pl.BlockSpec((d,), lambda i: (0,)),
            pl.BlockSpec((d,), lambda i: (0,)),
            pl.BlockSpec((d,), lambda i: (0,)),
            pl.BlockSpec((2, d), lambda i: (0, 0)),
        ],
        out_specs=pl.BlockSpec((blk, d), lambda i: (i, 0)),
        out_shape=jax.ShapeDtypeStruct((n, d), jnp.float32),
    )(agg, dinv, b, gamma, beta, stats)


@jax.jit
def kernel(x, edge_index, W, b, gamma, beta):
    n = x.shape[0]
    e = edge_index.shape[1]
    src = jnp.pad(edge_index[0], (0, _EP - e))
    dst = jnp.pad(edge_index[1], (0, _EP - e), constant_values=_NP)
    xp = jnp.pad(x, ((0, _NP - n), (0, 0)))
    parts = _deg(dst)
    h = _matmul(xp, W)
    g, dinv = _scale(h, parts)
    aggp = _agg(g, src, dst)
    return _bn_relu(aggp, dinv, b, gamma, beta, n)


# speedup vs baseline: 1.6219x; 1.0160x over previous
import functools
import jax
import jax.numpy as jnp
from jax import lax
from jax.experimental import pallas as pl
from jax.experimental.pallas import tpu as pltpu
from jax.experimental.pallas import tpu_sc as plsc

_NP = 10240
_NPH = 10368
_CS = 2560
_B = 64
_BSH = 6
_ETILE = 2048
_ED = 1024
_RING = 32
_EP = 163840

_MESH = plsc.VectorSubcoreMesh(core_axis_name="c", subcore_axis_name="s")
_SC_PARAMS = pltpu.CompilerParams(needs_layout_passes=False)


def _deg_body(dst_hbm, zeros_hbm, out_hbm, hist, dtile):
    core = lax.axis_index("c")
    sub = lax.axis_index("s")
    wid = core * 16 + sub
    e_sub = dst_hbm.shape[0] // 32
    n_tiles = e_sub // _ED
    ones16 = jnp.full((16,), 1.0, jnp.float32)
    ebase = wid * e_sub
    pltpu.sync_copy(zeros_hbm, hist)

    @pl.loop(0, n_tiles)
    def _(t):
        pltpu.sync_copy(dst_hbm.at[pl.ds(ebase + t * _ED, _ED)], dtile)

        @pl.loop(0, _ED // 16)
        def _(v):
            plsc.addupdate_scatter(hist, [dtile[pl.ds(v * 16, 16)]], ones16)

    pltpu.sync_copy(hist, out_hbm.at[wid])


def _deg(dst):
    zeros = jnp.zeros((_NPH,), jnp.float32)
    f = pl.kernel(
        _deg_body,
        out_type=jax.ShapeDtypeStruct((32, _NPH), jnp.float32),
        mesh=_MESH,
        scratch_types=[
            pltpu.VMEM((_NPH,), jnp.float32),
            pltpu.VMEM((_ED,), jnp.int32),
        ],
        compiler_params=_SC_PARAMS,
    )
    return f(dst, zeros)


def _agg_body(g_hbm, src_hbm, dst_hbm, out_hbm,
              acc, stile, dtile, srcb, dstb, rows):
    core = lax.axis_index("c")
    sub = lax.axis_index("s")
    e_sub = src_hbm.shape[0] // 16
    n_tiles = e_sub // _ETILE
    n_vecs = _ETILE // 16
    rstripe = _CS // 16
    iota16 = lax.iota(jnp.int32, 16)

    for p in range(_NP // _CS // 2):
        chunk = core + 2 * p
        base = chunk * _CS
        pltpu.sync_copy(g_hbm.at[pl.ds(base + sub * rstripe, rstripe)],
                        acc.at[pl.ds(sub * rstripe, rstripe)])
        plsc.subcore_barrier()

        ebase = sub * e_sub

        def batch_body(j, carry):
            jr = j & (_RING - 1)
            pltpu.sync_copy(g_hbm.at[srcb.at[jr]], rows)
            pltpu.sync_copy(rows, acc.at[dstb.at[jr]], add=True)
            return carry

        def tile_body(t, carry):
            cnt, jdone = carry
            pltpu.sync_copy(src_hbm.at[pl.ds(ebase + t * _ETILE, _ETILE)], stile)
            pltpu.sync_copy(dst_hbm.at[pl.ds(ebase + t * _ETILE, _ETILE)], dtile)

            def vec_body(v, cnt):
                s16 = stile[pl.ds(v * 16, 16)]
                d16 = dtile[pl.ds(v * 16, 16)]
                dloc = d16 - base
                m = (dloc >= 0) & (dloc < _CS)
                mi = m.astype(jnp.int32)
                pos = cnt + plsc.cumsum(mi) - 1
                plsc.store_scatter(
                    srcb, [(pos >> _BSH) & (_RING - 1), pos & (_B - 1)],
                    s16, mask=m)
                plsc.store_scatter(
                    dstb, [(pos >> _BSH) & (_RING - 1), pos & (_B - 1)],
                    dloc, mask=m)
                return cnt + jnp.sum(mi)

            cnt = lax.fori_loop(0, n_vecs, vec_body, cnt)
            jdone = lax.fori_loop(jdone, cnt >> _BSH, batch_body, jdone)
            return cnt, cnt >> _BSH

        cnt, jdone = lax.fori_loop(0, n_tiles, tile_body,
                                   (jnp.int32(0), jnp.int32(0)))

        for k in range(_B // 16):
            posk = cnt + k * 16 + iota16
            plsc.store_scatter(
                srcb, [(posk >> _BSH) & (_RING - 1), posk & (_B - 1)],
                jnp.zeros((16,), jnp.int32))
            plsc.store_scatter(
                dstb, [(posk >> _BSH) & (_RING - 1), posk & (_B - 1)],
                jnp.full((16,), _CS, jnp.int32))

        nb = (cnt + (_B - 1)) >> _BSH
        lax.fori_loop(jdone, nb, batch_body, jdone)
        plsc.subcore_barrier()
        pltpu.sync_copy(acc.at[pl.ds(sub * rstripe, rstripe)],
                        out_hbm.at[pl.ds(base + sub * rstripe, rstripe)])


def _agg(g, src, dst):
    f = pl.kernel(
        _agg_body,
        out_type=jax.ShapeDtypeStruct((_NP, 4, 128), jnp.float32),
        mesh=_MESH,
        scratch_types=[
            pltpu.VMEM_SHARED((_CS + 8, 4, 128), jnp.float32),
            pltpu.VMEM((_ETILE,), jnp.int32),
            pltpu.VMEM((_ETILE,), jnp.int32),
            pltpu.VMEM((_RING, _B), jnp.int32),
            pltpu.VMEM((_RING, _B), jnp.int32),
            pltpu.VMEM((_B, 4, 128), jnp.float32),
        ],
        compiler_params=_SC_PARAMS,
    )
    return f(g.reshape(_NP, 4, 128), src, dst).reshape(_NP, 512)


def _matmul_body(x_ref, w_ref, out_ref):
    out_ref[...] = jax.lax.dot_general(
        x_ref[...], w_ref[...], (((1,), (0,)), ((), ())),
        preferred_element_type=jnp.float32,
        precision=jax.lax.Precision.HIGHEST,
    )


def _matmul(x, W):
    n, d_in = x.shape
    d_out = W.shape[1]
    blk = 2048
    return pl.pallas_call(
        _matmul_body,
        grid=(n // blk,),
        in_specs=[
            pl.BlockSpec((blk, d_in), lambda i: (i, 0)),
            pl.BlockSpec((d_in, d_out), lambda i: (0, 0)),
        ],
        out_specs=pl.BlockSpec((blk, d_out), lambda i: (i, 0)),
        out_shape=jax.ShapeDtypeStruct((n, d_out), jnp.float32),
    )(x, W)


def _scale_body(h_ref, parts_ref, g_ref, dinv_ref):
    deg = jnp.sum(parts_ref[...], axis=0) + 1.0
    dinv = jax.lax.rsqrt(deg).reshape(-1, 1)
    dinv_ref[...] = dinv
    g_ref[...] = h_ref[...] * dinv


def _scale(h, parts):
    n, d = h.shape
    blk = 2048
    return pl.pallas_call(
        _scale_body,
        grid=(n // blk,),
        in_specs=[
            pl.BlockSpec((blk, d), lambda i: (i, 0)),
            pl.BlockSpec((32, blk), lambda i: (0, i)),
        ],
        out_specs=[
            pl.BlockSpec((blk, d), lambda i: (i, 0)),
            pl.BlockSpec((blk, 1), lambda i: (i, 0)),
        ],
        out_shape=[
            jax.ShapeDtypeStruct((n, d), jnp.float32),
            jax.ShapeDtypeStruct((n, 1), jnp.float32),
        ],
    )(h, parts)


def _bn_stats_body(acc_ref, dinv_ref, b_ref, stats_ref):
    i = pl.program_id(0)
    y = acc_ref[...] * dinv_ref[...] + b_ref[...].reshape(1, -1)
    s = jnp.sum(y, axis=0, keepdims=True)
    ss = jnp.sum(y * y, axis=0, keepdims=True)
    blk_stats = jnp.concatenate([s, ss], axis=0)

    @pl.when(i == 0)
    def _():
        stats_ref[...] = blk_stats

    @pl.when(i > 0)
    def _():
        stats_ref[...] += blk_stats


def _bn_apply_body(acc_ref, dinv_ref, b_ref, gamma_ref, beta_ref, stats_ref,
                   out_ref, *, n):
    y = acc_ref[...] * dinv_ref[...] + b_ref[...].reshape(1, -1)
    mean = stats_ref[0, :].reshape(1, -1) / n
    var = stats_ref[1, :].reshape(1, -1) / n - mean * mean
    z = (y - mean) * jax.lax.rsqrt(var + 1e-5)
    out_ref[...] = jnp.maximum(
        gamma_ref[...].reshape(1, -1) * z + beta_ref[...].reshape(1, -1), 0.0
    )


def _bn_relu(agg, dinv, b, gamma, beta, n):
    d = agg.shape[1]
    blk = 2000
    grid = (n // blk,)
    stats = pl.pallas_call(
        _bn_stats_body,
        grid=grid,
        in_specs=[
            pl.BlockSpec((blk, d), lambda i: (i, 0)),
            pl.BlockSpec((blk, 1), lambda i: (i, 0)),
            pl.BlockSpec((d,), lambda i: (0,)),
        ],
        out_specs=pl.BlockSpec((2, d), lambda i: (0, 0)),
        out_shape=jax.ShapeDtypeStruct((2, d), jnp.float32),
    )(agg, dinv, b)
    return pl.pallas_call(
        functools.partial(_bn_apply_body, n=float(n)),
        grid=grid,
        in_specs=[
            pl.BlockSpec((blk, d), lambda i: (i, 0)),
            pl.BlockSpec((blk, 1), lambda i: (i, 0)),
            pl.BlockSpec((d,), lambda i: (0,)),
            pl.BlockSpec((d,), lambda i: (0,)),
            pl.BlockSpec((d,), lambda i: (0,)),
            pl.BlockSpec((2, d), lambda i: (0, 0)),
        ],
        out_specs=pl.BlockSpec((blk, d), lambda i: (i, 0)),
        out_shape=jax.ShapeDtypeStruct((n, d), jnp.float32),
    )(agg, dinv, b, gamma, beta, stats)


@jax.jit
def kernel(x, edge_index, W, b, gamma, beta):
    n = x.shape[0]
    e = edge_index.shape[1]
    src = jnp.pad(edge_index[0], (0, _EP - e))
    dst = jnp.pad(edge_index[1], (0, _EP - e), constant_values=_NP)
    xp = jnp.pad(x, ((0, _NP - n), (0, 0)))
    parts = _deg(dst)
    h = _matmul(xp, W)
    g, dinv = _scale(h, parts)
    aggp = _agg(g, src, dst)
    return _bn_relu(aggp, dinv, b, gamma, beta, n)
